# baseline (speedup 1.0000x reference)
import functools

import jax
import jax.numpy as jnp
from jax import lax
from jax.experimental import pallas as pl
from jax.experimental.pallas import tpu as pltpu
from jax.experimental.pallas import tpu_sc as plsc

B, S, D = 4, 4096, 1024
NC, NS, L = 2, 16, 16
NW = NC * NS
SPW = S // NW
T = 16
NCH = SPW // T
NSTEP = NCH * B
NVEC = T * D // L
NXB = 5
LOOKAHEAD = 4

_mesh = plsc.VectorSubcoreMesh(
    core_axis_name="c", subcore_axis_name="s", num_cores=NC, num_subcores=NS
)


@functools.partial(
    pl.kernel,
    out_type=jax.ShapeDtypeStruct((B, S, D), jnp.float32),
    mesh=_mesh,
    compiler_params=pltpu.CompilerParams(use_tc_tiling_on_sc=True),
    scratch_types=[
        [pltpu.VMEM((T, D), jnp.float32)] * 2,
        [pltpu.VMEM((T, D), jnp.float32)] * NXB,
        [pltpu.SemaphoreType.DMA] * 2,
        [pltpu.SemaphoreType.DMA] * NXB,
        [pltpu.SemaphoreType.DMA] * NXB,
    ],
)
def _sc_add(x_hbm, pos_hbm, out_hbm, p_v, x_v, sem_p, sem_x, sem_o):
    wid = lax.axis_index("s") * NC + lax.axis_index("c")
    s0 = wid * SPW

    def pos_load(ci):
        return pltpu.make_async_copy(
            pos_hbm.at[pl.ds(s0 + ci * T, T)], p_v[ci % 2], sem_p[ci % 2]
        )

    def x_load(step):
        ci, b = step // B, step % B
        return pltpu.make_async_copy(
            x_hbm.at[b, pl.ds(s0 + ci * T, T)], x_v[step % NXB], sem_x[step % NXB]
        )

    def out_store(step):
        ci, b = step // B, step % B
        return pltpu.make_async_copy(
            x_v[step % NXB], out_hbm.at[b, pl.ds(s0 + ci * T, T)], sem_o[step % NXB]
        )

    pos_load(0).start()
    for s in range(LOOKAHEAD):
        x_load(s).start()

    for step in range(NSTEP):
        ci = step // B
        if step % B == 0:
            pos_load(ci).wait()
            if ci + 1 < NCH:
                pos_load(ci + 1).start()
        if step + LOOKAHEAD < NSTEP:
            prev = step + LOOKAHEAD - NXB
            if prev >= 0:
                out_store(prev).wait()
            x_load(step + LOOKAHEAD).start()
        x_load(step).wait()

        pv = p_v[ci % 2]
        xv = x_v[step % NXB]

        @plsc.parallel_loop(0, T)
        def _row(r):
            @plsc.parallel_loop(0, D // L, unroll=8)
            def _acc(c):
                sl = pl.ds(c * L, L)
                plsc.addupdate(xv.at[r, sl], pv[r, sl])

        out_store(step).start()

    for step in range(NSTEP - NXB, NSTEP):
        out_store(step).wait()


def kernel(x, pos_table):
    return _sc_add(x, pos_table)


# speedup vs baseline: 1.0665x; 1.0665x over previous
import functools

import jax
import jax.numpy as jnp
from jax import lax
from jax.experimental import pallas as pl
from jax.experimental.pallas import tpu as pltpu
from jax.experimental.pallas import tpu_sc as plsc

B, S, D = 4, 4096, 1024
NC, NS, L = 2, 16, 16
NW = NC * NS
SPW = S // NW
T = 16
NCH = SPW // T
NSTEP = NCH * B
NVEC = T * D // L
NXB = 4
LOOKAHEAD = 3

_mesh = plsc.VectorSubcoreMesh(
    core_axis_name="c", subcore_axis_name="s", num_cores=NC, num_subcores=NS
)


@functools.partial(
    pl.kernel,
    out_type=jax.ShapeDtypeStruct((B, S, D), jnp.float32),
    mesh=_mesh,
    compiler_params=pltpu.CompilerParams(use_tc_tiling_on_sc=True),
    scratch_types=[
        [pltpu.VMEM((T, D), jnp.float32)] * 2,
        [pltpu.VMEM((T, D), jnp.float32)] * NXB,
        [pltpu.SemaphoreType.DMA] * 2,
        [pltpu.SemaphoreType.DMA] * NXB,
        [pltpu.SemaphoreType.DMA] * NXB,
    ],
)
def _sc_add(x_hbm, pos_hbm, out_hbm, p_v, x_v, sem_p, sem_x, sem_o):
    wid = lax.axis_index("s") * NC + lax.axis_index("c")
    s0 = wid * SPW

    def pos_load(ci):
        return pltpu.make_async_copy(
            pos_hbm.at[pl.ds(s0 + ci * T, T)], p_v[ci % 2], sem_p[ci % 2]
        )

    def x_load(step):
        ci, b = step // B, step % B
        return pltpu.make_async_copy(
            x_hbm.at[b, pl.ds(s0 + ci * T, T)], x_v[step % NXB], sem_x[step % NXB]
        )

    def out_store(step):
        ci, b = step // B, step % B
        return pltpu.make_async_copy(
            x_v[step % NXB], out_hbm.at[b, pl.ds(s0 + ci * T, T)], sem_o[step % NXB]
        )

    pos_load(0).start()
    for s in range(LOOKAHEAD):
        x_load(s).start()

    for step in range(NSTEP):
        ci = step // B
        if step % B == 0:
            pos_load(ci).wait()
            if ci + 1 < NCH:
                pos_load(ci + 1).start()
        if step + LOOKAHEAD < NSTEP:
            prev = step + LOOKAHEAD - NXB
            if prev >= 0:
                out_store(prev).wait()
            x_load(step + LOOKAHEAD).start()
        x_load(step).wait()

        pv = p_v[ci % 2]
        xv = x_v[step % NXB]

        @plsc.parallel_loop(0, NVEC, unroll=8)
        def _acc(i):
            r = i // (D // L)
            c = (i % (D // L)) * L
            sl = pl.ds(c, L)
            plsc.addupdate(xv.at[r, sl], pv[r, sl])

        out_store(step).start()

    for step in range(NSTEP - NXB, NSTEP):
        out_store(step).wait()


def kernel(x, pos_table):
    return _sc_add(x, pos_table)


# speedup vs baseline: 1.1511x; 1.0793x over previous
import functools

import jax
import jax.numpy as jnp
from jax import lax
from jax.experimental import pallas as pl
from jax.experimental.pallas import tpu as pltpu
from jax.experimental.pallas import tpu_sc as plsc

B, S, D = 4, 4096, 1024
NC, NS, L = 2, 16, 16
NW = NC * NS
SPW = S // NW
T = 16
NCH = SPW // T
NSTEP = NCH * B
NVEC = T * D // L
NXB = 4
LOOKAHEAD = 3

_mesh = plsc.VectorSubcoreMesh(
    core_axis_name="c", subcore_axis_name="s", num_cores=NC, num_subcores=NS
)


@functools.partial(
    pl.kernel,
    out_type=jax.ShapeDtypeStruct((B, S, D), jnp.float32),
    mesh=_mesh,
    compiler_params=pltpu.CompilerParams(use_tc_tiling_on_sc=True),
    scratch_types=[
        [pltpu.VMEM((T, D), jnp.float32)] * 2,
        [pltpu.VMEM((T, D), jnp.float32)] * NXB,
        [pltpu.SemaphoreType.DMA] * 2,
        [pltpu.SemaphoreType.DMA] * NXB,
        [pltpu.SemaphoreType.DMA] * NXB,
    ],
)
def _sc_add(x_hbm, pos_hbm, out_hbm, p_v, x_v, sem_p, sem_x, sem_o):
    wid = lax.axis_index("s") * NC + lax.axis_index("c")
    s0 = wid * SPW

    def pos_load(ci):
        return pltpu.make_async_copy(
            pos_hbm.at[pl.ds(s0 + ci * T, T)], p_v[ci % 2], sem_p[ci % 2]
        )

    def x_load(step):
        ci, b = step // B, step % B
        return pltpu.make_async_copy(
            x_hbm.at[b, pl.ds(s0 + ci * T, T)], x_v[step % NXB], sem_x[step % NXB]
        )

    def out_store(step):
        ci, b = step // B, step % B
        return pltpu.make_async_copy(
            x_v[step % NXB], out_hbm.at[b, pl.ds(s0 + ci * T, T)], sem_o[step % NXB]
        )

    pos_load(0).start()
    for s in range(LOOKAHEAD):
        x_load(s).start()

    for step in range(NSTEP):
        ci = step // B
        if step % B == 0:
            pos_load(ci).wait()
            if ci + 1 < NCH:
                pos_load(ci + 1).start()
        x_load(step).wait()

        pv = p_v[ci % 2]
        xv = x_v[step % NXB]

        @plsc.parallel_loop(0, NVEC, unroll=8)
        def _acc(i):
            r = i // (D // L)
            c = (i % (D // L)) * L
            sl = pl.ds(c, L)
            plsc.addupdate(xv.at[r, sl], pv[r, sl])

        out_store(step).start()
        if step + LOOKAHEAD < NSTEP:
            prev = step + LOOKAHEAD - NXB
            if prev >= 0:
                out_store(prev).wait()
            x_load(step + LOOKAHEAD).start()

    for step in range(NSTEP - NXB, NSTEP):
        out_store(step).wait()


def kernel(x, pos_table):
    return _sc_add(x, pos_table)


# speedup vs baseline: 1.1600x; 1.0077x over previous
import functools

import jax
import jax.numpy as jnp
from jax import lax
from jax.experimental import pallas as pl
from jax.experimental.pallas import tpu as pltpu
from jax.experimental.pallas import tpu_sc as plsc

B, S, D = 4, 4096, 1024
NC, NS, L = 2, 16, 16
NW = NC * NS
SPW = S // NW
T = 16
NCH = SPW // T
NSTEP = NCH * B
NVEC = T * D // L
NXB = 5
LOOKAHEAD = 4

_mesh = plsc.VectorSubcoreMesh(
    core_axis_name="c", subcore_axis_name="s", num_cores=NC, num_subcores=NS
)


@functools.partial(
    pl.kernel,
    out_type=jax.ShapeDtypeStruct((B, S, D), jnp.float32),
    mesh=_mesh,
    compiler_params=pltpu.CompilerParams(use_tc_tiling_on_sc=True),
    scratch_types=[
        [pltpu.VMEM((T, D), jnp.float32)] * 2,
        [pltpu.VMEM((T, D), jnp.float32)] * NXB,
        [pltpu.SemaphoreType.DMA] * 2,
        [pltpu.SemaphoreType.DMA] * NXB,
        [pltpu.SemaphoreType.DMA] * NXB,
    ],
)
def _sc_add(x_hbm, pos_hbm, out_hbm, p_v, x_v, sem_p, sem_x, sem_o):
    wid = lax.axis_index("s") * NC + lax.axis_index("c")
    s0 = wid * SPW

    def pos_load(ci):
        return pltpu.make_async_copy(
            pos_hbm.at[pl.ds(s0 + ci * T, T)], p_v[ci % 2], sem_p[ci % 2]
        )

    def x_load(step):
        ci, b = step // B, step % B
        return pltpu.make_async_copy(
            x_hbm.at[b, pl.ds(s0 + ci * T, T)], x_v[step % NXB], sem_x[step % NXB]
        )

    def out_store(step):
        ci, b = step // B, step % B
        return pltpu.make_async_copy(
            x_v[step % NXB], out_hbm.at[b, pl.ds(s0 + ci * T, T)], sem_o[step % NXB]
        )

    pos_load(0).start()
    for s in range(LOOKAHEAD):
        x_load(s).start()

    for step in range(NSTEP):
        ci = step // B
        if step % B == 0:
            pos_load(ci).wait()
            if ci + 1 < NCH:
                pos_load(ci + 1).start()
        x_load(step).wait()

        pv = p_v[ci % 2]
        xv = x_v[step % NXB]

        @plsc.parallel_loop(0, NVEC, unroll=8)
        def _acc(i):
            r = i // (D // L)
            c = (i % (D // L)) * L
            sl = pl.ds(c, L)
            plsc.addupdate(xv.at[r, sl], pv[r, sl])

        out_store(step).start()
        if step + LOOKAHEAD < NSTEP:
            prev = step + LOOKAHEAD - NXB
            if prev >= 0:
                out_store(prev).wait()
            x_load(step + LOOKAHEAD).start()

    for step in range(NSTEP - NXB, NSTEP):
        out_store(step).wait()


def kernel(x, pos_table):
    return _sc_add(x, pos_table)
